# baseline (speedup 1.0000x reference)
import numpy as np
import jax
import jax.numpy as jnp
from jax.experimental import pallas as pl

_STRIDES = (8, 16, 32, 64)
_SIZES = (32, 64, 128, 256)
_HW = (64, 32, 16, 8)
_RATIOS = np.array([0.5, 1.0, 2.0])
_SCALES = np.array([1.0, 2.0 ** (1.0 / 3.0), 2.0 ** (2.0 / 3.0)])
_A = 9
_LANES = 128
_N_ANCH = sum(h * h * _A for h in _HW)
_N_ROWS = _N_ANCH * 4 // _LANES


def _wh_table(box_size):
    anchors = box_size * np.tile(_SCALES, (2, len(_RATIOS))).T
    areas = anchors[:, 0] * anchors[:, 1]
    anchors[:, 0] = np.sqrt(areas * np.repeat(_RATIOS, len(_SCALES)))
    anchors[:, 1] = anchors[:, 0] / np.repeat(_RATIOS, len(_SCALES))
    return anchors.astype(np.float32)


def _gen_body(xywh_ref, xyxy_ref):
    lane = jax.lax.broadcasted_iota(jnp.int32, (1, _LANES), 1)
    c = lane % 4
    is_cx = c == 0
    is_cy = c == 1
    is_w = c == 2
    is_x = (c % 4) % 2 == 0

    row_off = 0
    for hw, stride, size in zip(_HW, _STRIDES, _SIZES):
        s = float(stride)
        n_rows = hw * hw * _A * 4 // _LANES
        tab = _wh_table(size)

        r = jax.lax.broadcasted_iota(jnp.int32, (n_rows, 1), 0)
        g = r * _LANES + lane
        i = g >> 2
        site = i // _A
        a = i - site * _A
        x = site & (hw - 1)
        y = site >> hw.bit_length() - 1

        cx = (x.astype(jnp.float32) + 0.5) * s
        cy = (y.astype(jnp.float32) + 0.5) * s
        wa = jnp.full(g.shape, float(tab[0, 0]), jnp.float32)
        ha = jnp.full(g.shape, float(tab[0, 1]), jnp.float32)
        for k in range(1, _A):
            sel = a == k
            wa = jnp.where(sel, float(tab[k, 0]), wa)
            ha = jnp.where(sel, float(tab[k, 1]), ha)

        xywh = jnp.where(is_cx, cx, jnp.where(is_cy, cy, jnp.where(is_w, wa, ha)))
        ctr = jnp.where(is_x, cx, cy)
        half = jnp.where(is_x, wa, ha) * 0.5
        xyxy = jnp.where(c < 2, ctr - half, ctr + half)

        xywh_ref[pl.ds(row_off, n_rows), :] = xywh
        xyxy_ref[pl.ds(row_off, n_rows), :] = xyxy
        row_off += n_rows


def _generate():
    out_shape = (
        jax.ShapeDtypeStruct((_N_ROWS, _LANES), jnp.float32),
        jax.ShapeDtypeStruct((_N_ROWS, _LANES), jnp.float32),
    )
    return pl.pallas_call(_gen_body, out_shape=out_shape)()


def kernel(feat0, feat1, feat2, feat3):
    xywh, xyxy = _generate()
    return (xywh.reshape(_N_ANCH, 4), xyxy.reshape(_N_ANCH, 4))


# speedup vs baseline: 22.3429x; 22.3429x over previous
import numpy as np
import jax
import jax.numpy as jnp
from jax.experimental import pallas as pl

_STRIDES = (8, 16, 32, 64)
_SIZES = (32, 64, 128, 256)
_HW = (64, 32, 16, 8)
_RATIOS = np.array([0.5, 1.0, 2.0])
_SCALES = np.array([1.0, 2.0 ** (1.0 / 3.0), 2.0 ** (2.0 / 3.0)])
_A = 9
_N_ANCH = sum(h * h * _A for h in _HW)


def _wh_table(box_size):
    anchors = box_size * np.tile(_SCALES, (2, len(_RATIOS))).T
    areas = anchors[:, 0] * anchors[:, 1]
    anchors[:, 0] = np.sqrt(areas * np.repeat(_RATIOS, len(_SCALES)))
    anchors[:, 1] = anchors[:, 0] / np.repeat(_RATIOS, len(_SCALES))
    return anchors.astype(np.float32)


def _gen_body(xywh_ref, xyxy_ref):
    c = jax.lax.broadcasted_iota(jnp.int32, (4, 1), 0)
    is_cx = c == 0
    is_cy = c == 1
    is_w = c == 2
    is_x = c % 2 == 0

    off = 0
    for hw, stride, size in zip(_HW, _STRIDES, _SIZES):
        s = float(stride)
        n = hw * hw * _A
        tab = _wh_table(size)

        i = jax.lax.broadcasted_iota(jnp.int32, (1, n), 1)
        site = i // _A
        a = i - site * _A
        x = site & (hw - 1)
        y = site >> hw.bit_length() - 1

        cx = (x.astype(jnp.float32) + 0.5) * s
        cy = (y.astype(jnp.float32) + 0.5) * s
        wa = jnp.full(i.shape, float(tab[0, 0]), jnp.float32)
        ha = jnp.full(i.shape, float(tab[0, 1]), jnp.float32)
        for k in range(1, _A):
            sel = a == k
            wa = jnp.where(sel, float(tab[k, 0]), wa)
            ha = jnp.where(sel, float(tab[k, 1]), ha)

        xywh = jnp.where(is_cx, cx, jnp.where(is_cy, cy, jnp.where(is_w, wa, ha)))
        ctr = jnp.where(is_x, cx, cy)
        half = jnp.where(is_x, wa, ha) * 0.5
        xyxy = jnp.where(c < 2, ctr - half, ctr + half)

        xywh_ref[:, pl.ds(off, n)] = xywh
        xyxy_ref[:, pl.ds(off, n)] = xyxy
        off += n


def _generate():
    out_shape = (
        jax.ShapeDtypeStruct((4, _N_ANCH), jnp.float32),
        jax.ShapeDtypeStruct((4, _N_ANCH), jnp.float32),
    )
    return pl.pallas_call(_gen_body, out_shape=out_shape)()


def kernel(feat0, feat1, feat2, feat3):
    xywh_t, xyxy_t = _generate()
    return (xywh_t.T, xyxy_t.T)
